# baseline (speedup 1.0000x reference)
import functools

import jax
import jax.numpy as jnp
import numpy as np
from jax import lax
from jax.experimental import pallas as pl
from jax.experimental.pallas import tpu as pltpu
from jax.experimental.pallas import tpu_sc as plsc

B = 4096
NROW = 72
D = 256
NSEL = 6
NC = 2
NS = 16
NW = NC * NS
PER_W = B // NW
IDX_PER_W = NSEL * PER_W
RPC = 64
CPP = PER_W // RPC
NCHUNK = NSEL * CPP
NSLOT = 4

_HANDS = (27, 28, 29, 39, 40, 41)

_mesh = plsc.VectorSubcoreMesh(core_axis_name="c", subcore_axis_name="s")


@functools.partial(
    pl.kernel,
    out_type=jax.ShapeDtypeStruct((NSEL, B, D), jnp.float32),
    mesh=_mesh,
    scratch_types=[
        pltpu.VMEM((IDX_PER_W,), jnp.int32),
        pltpu.VMEM((NSLOT, RPC, D), jnp.float32),
        pltpu.SemaphoreType.DMA,
        pltpu.SemaphoreType.DMA,
        pltpu.SemaphoreType.DMA,
        pltpu.SemaphoreType.DMA,
        pltpu.SemaphoreType.DMA,
        pltpu.SemaphoreType.DMA,
        pltpu.SemaphoreType.DMA,
        pltpu.SemaphoreType.DMA,
    ],
)
def _select_hands(x_hbm, out_hbm, idx_v, rowbuf,
                  sg0, sg1, sg2, sg3, so0, so1, so2, so3):
    wid = lax.axis_index("s") * NC + lax.axis_index("c")
    wb = wid * PER_W
    sems_g = (sg0, sg1, sg2, sg3)
    sems_o = (so0, so1, so2, so3)

    ramp = lax.iota(jnp.int32, 16)
    for j in range(NSEL):
        for k in range(PER_W // 16):
            idx_v[pl.ds((j * (PER_W // 16) + k) * 16, 16)] = (
                (wb + k * 16 + ramp) * NROW + _HANDS[j]
            )

    def start_gather(c, slot):
        return pltpu.async_copy(
            x_hbm.at[idx_v.at[pl.ds(c * RPC, RPC)]],
            rowbuf.at[slot],
            sems_g[slot],
        )

    def start_out(c, slot):
        j, q = divmod(c, CPP)
        return pltpu.async_copy(
            rowbuf.at[slot],
            out_hbm.at[j, pl.ds(wb + q * RPC, RPC)],
            sems_o[slot],
        )

    DEPTH = NSLOT - 1
    gather_h = [None] * NSLOT
    out_h = [None] * NSLOT
    for c in range(DEPTH):
        gather_h[c] = start_gather(c, c)
    for c in range(NCHUNK):
        slot = c % NSLOT
        nslot = (c + DEPTH) % NSLOT
        if c + DEPTH < NCHUNK:
            if out_h[nslot] is not None:
                out_h[nslot].wait()
            gather_h[nslot] = start_gather(c + DEPTH, nslot)
        gather_h[slot].wait()
        out_h[slot] = start_out(c, slot)
    for h in out_h:
        if h is not None:
            h.wait()


def kernel(input):
    x2d = input.reshape(B * NROW, D)
    planes = _select_hands(x2d)
    return jnp.swapaxes(planes, 0, 1)


# speedup vs baseline: 1.0024x; 1.0024x over previous
import functools

import jax
import jax.numpy as jnp
import numpy as np
from jax import lax
from jax.experimental import pallas as pl
from jax.experimental.pallas import tpu as pltpu
from jax.experimental.pallas import tpu_sc as plsc

B = 4096
NROW = 72
D = 256
NSEL = 6
NC = 2
NS = 16
NW = NC * NS
PER_W = B // NW
IDX_PER_W = NSEL * PER_W
NSLOT = 3

_HANDS = (27, 28, 29, 39, 40, 41)

_mesh = plsc.VectorSubcoreMesh(core_axis_name="c", subcore_axis_name="s")


@functools.partial(
    pl.kernel,
    out_type=jax.ShapeDtypeStruct((NSEL, B, D), jnp.float32),
    mesh=_mesh,
    scratch_types=[
        pltpu.VMEM((IDX_PER_W,), jnp.int32),
        pltpu.VMEM((NSLOT, PER_W, D), jnp.float32),
        pltpu.SemaphoreType.DMA,
        pltpu.SemaphoreType.DMA,
        pltpu.SemaphoreType.DMA,
        pltpu.SemaphoreType.DMA,
        pltpu.SemaphoreType.DMA,
        pltpu.SemaphoreType.DMA,
    ],
    compiler_params=pltpu.CompilerParams(
        disable_bounds_checks=True,
        disable_semaphore_checks=True,
    ),
)
def _select_hands(x_hbm, out_hbm, idx_v, rowbuf,
                  sg0, sg1, sg2, so0, so1, so2):
    wid = lax.axis_index("s") * NC + lax.axis_index("c")
    wb = wid * PER_W
    sems_g = (sg0, sg1, sg2)
    sems_o = (so0, so1, so2)

    ramp = lax.iota(jnp.int32, 16)
    for j in range(NSEL):
        for k in range(PER_W // 16):
            idx_v[pl.ds((j * (PER_W // 16) + k) * 16, 16)] = (
                (wb + k * 16 + ramp) * NROW + _HANDS[j]
            )

    def start_gather(j, slot):
        return pltpu.async_copy(
            x_hbm.at[idx_v.at[pl.ds(j * PER_W, PER_W)]],
            rowbuf.at[slot],
            sems_g[slot],
        )

    def start_out(j, slot):
        return pltpu.async_copy(
            rowbuf.at[slot],
            out_hbm.at[j, pl.ds(wb, PER_W)],
            sems_o[slot],
        )

    gather_h = [None] * NSLOT
    out_h = [None] * NSLOT
    gather_h[0] = start_gather(0, 0)
    gather_h[1] = start_gather(1, 1)
    for j in range(NSEL):
        slot = j % NSLOT
        nslot = (j + 2) % NSLOT
        if j + 2 < NSEL:
            if out_h[nslot] is not None:
                out_h[nslot].wait()
            gather_h[nslot] = start_gather(j + 2, nslot)
        gather_h[slot].wait()
        out_h[slot] = start_out(j, slot)
    for h in out_h:
        h.wait()


def kernel(input):
    x2d = input.reshape(B * NROW, D)
    planes = _select_hands(x2d)
    return jnp.swapaxes(planes, 0, 1)


# speedup vs baseline: 1.0047x; 1.0023x over previous
import functools

import jax
import jax.numpy as jnp
import numpy as np
from jax import lax
from jax.experimental import pallas as pl
from jax.experimental.pallas import tpu as pltpu
from jax.experimental.pallas import tpu_sc as plsc

B = 4096
NROW = 72
D = 256
NSEL = 6
NC = 2
NS = 16
NW = NC * NS
PER_W = B // NW
IDX_PER_W = NSEL * PER_W
NSLOT = 3

_HANDS = (27, 28, 29, 39, 40, 41)

_mesh = plsc.VectorSubcoreMesh(core_axis_name="c", subcore_axis_name="s")


@functools.partial(
    pl.kernel,
    out_type=jax.ShapeDtypeStruct((NSEL, B, D), jnp.float32),
    mesh=_mesh,
    scratch_types=[
        pltpu.VMEM((IDX_PER_W,), jnp.int32),
        pltpu.VMEM((NSLOT, PER_W, D), jnp.float32),
        pltpu.SemaphoreType.DMA,
        pltpu.SemaphoreType.DMA,
        pltpu.SemaphoreType.DMA,
        pltpu.SemaphoreType.DMA,
        pltpu.SemaphoreType.DMA,
        pltpu.SemaphoreType.DMA,
    ],
)
def _select_hands(x_hbm, out_hbm, idx_v, rowbuf,
                  sg0, sg1, sg2, so0, so1, so2):
    wid = lax.axis_index("s") * NC + lax.axis_index("c")
    wb = wid * PER_W
    sems_g = (sg0, sg1, sg2)
    sems_o = (so0, so1, so2)

    ramp = (wb + lax.iota(jnp.int32, 16)) * NROW
    def fill_idx(j):
        for k in range(PER_W // 16):
            idx_v[pl.ds((j * (PER_W // 16) + k) * 16, 16)] = (
                ramp + (k * 16 * NROW + _HANDS[j])
            )

    def start_gather(j, slot):
        return pltpu.async_copy(
            x_hbm.at[idx_v.at[pl.ds(j * PER_W, PER_W)]],
            rowbuf.at[slot],
            sems_g[slot],
        )

    def start_out(j, slot):
        return pltpu.async_copy(
            rowbuf.at[slot],
            out_hbm.at[j, pl.ds(wb, PER_W)],
            sems_o[slot],
        )

    gather_h = [None] * NSLOT
    out_h = [None] * NSLOT
    fill_idx(0)
    gather_h[0] = start_gather(0, 0)
    fill_idx(1)
    gather_h[1] = start_gather(1, 1)
    for j in range(NSEL):
        slot = j % NSLOT
        nslot = (j + 2) % NSLOT
        if j + 2 < NSEL:
            fill_idx(j + 2)
            if out_h[nslot] is not None:
                out_h[nslot].wait()
            gather_h[nslot] = start_gather(j + 2, nslot)
        gather_h[slot].wait()
        out_h[slot] = start_out(j, slot)
    for h in out_h:
        h.wait()


def kernel(input):
    x2d = input.reshape(B * NROW, D)
    planes = _select_hands(x2d)
    return jnp.swapaxes(planes, 0, 1)


# speedup vs baseline: 1.0069x; 1.0021x over previous
import functools

import jax
import jax.numpy as jnp
import numpy as np
from jax import lax
from jax.experimental import pallas as pl
from jax.experimental.pallas import tpu as pltpu
from jax.experimental.pallas import tpu_sc as plsc

B = 4096
NROW = 72
D = 256
NSEL = 6
NC = 2
NS = 16
NW = NC * NS
PER_W = B // NW
IDX_PER_W = NSEL * PER_W
NSLOT = 3

_HANDS = (27, 28, 29, 39, 40, 41)

_mesh = plsc.VectorSubcoreMesh(core_axis_name="c", subcore_axis_name="s")


@functools.partial(
    pl.kernel,
    out_type=jax.ShapeDtypeStruct((NSEL, B, D), jnp.float32),
    mesh=_mesh,
    scratch_types=[
        pltpu.VMEM((IDX_PER_W,), jnp.int32),
        pltpu.VMEM((NSLOT, PER_W, D), jnp.float32),
        pltpu.SemaphoreType.DMA,
        pltpu.SemaphoreType.DMA,
        pltpu.SemaphoreType.DMA,
        pltpu.SemaphoreType.DMA,
        pltpu.SemaphoreType.DMA,
        pltpu.SemaphoreType.DMA,
    ],
    compiler_params=pltpu.CompilerParams(skip_device_barrier=True),
)
def _select_hands(x_hbm, out_hbm, idx_v, rowbuf,
                  sg0, sg1, sg2, so0, so1, so2):
    wid = lax.axis_index("s") * NC + lax.axis_index("c")
    wb = wid * PER_W
    sems_g = (sg0, sg1, sg2)
    sems_o = (so0, so1, so2)

    ramp = (wb + lax.iota(jnp.int32, 16)) * NROW
    def fill_idx(j):
        for k in range(PER_W // 16):
            idx_v[pl.ds((j * (PER_W // 16) + k) * 16, 16)] = (
                ramp + (k * 16 * NROW + _HANDS[j])
            )

    def start_gather(j, slot):
        return pltpu.async_copy(
            x_hbm.at[idx_v.at[pl.ds(j * PER_W, PER_W)]],
            rowbuf.at[slot],
            sems_g[slot],
        )

    def start_out(j, slot):
        return pltpu.async_copy(
            rowbuf.at[slot],
            out_hbm.at[j, pl.ds(wb, PER_W)],
            sems_o[slot],
        )

    gather_h = [None] * NSLOT
    out_h = [None] * NSLOT
    fill_idx(0)
    gather_h[0] = start_gather(0, 0)
    fill_idx(1)
    gather_h[1] = start_gather(1, 1)
    for j in range(NSEL):
        slot = j % NSLOT
        nslot = (j + 2) % NSLOT
        if j + 2 < NSEL:
            fill_idx(j + 2)
            if out_h[nslot] is not None:
                out_h[nslot].wait()
            gather_h[nslot] = start_gather(j + 2, nslot)
        gather_h[slot].wait()
        out_h[slot] = start_out(j, slot)
    for h in out_h:
        h.wait()


def kernel(input):
    x2d = input.reshape(B * NROW, D)
    planes = _select_hands(x2d)
    return jnp.swapaxes(planes, 0, 1)


# speedup vs baseline: 1.0075x; 1.0007x over previous
import functools

import jax
import jax.numpy as jnp
import numpy as np
from jax import lax
from jax.experimental import pallas as pl
from jax.experimental.pallas import tpu as pltpu
from jax.experimental.pallas import tpu_sc as plsc

B = 4096
NROW = 72
D = 256
NSEL = 6
NC = 2
NS = 16
NW = NC * NS
PER_W = B // NW
IDX_PER_W = NSEL * PER_W
NSLOT = 3

_HANDS = (27, 28, 29, 39, 40, 41)

_mesh = plsc.VectorSubcoreMesh(core_axis_name="c", subcore_axis_name="s")


@functools.partial(
    pl.kernel,
    out_type=jax.ShapeDtypeStruct((NSEL, B, D), jnp.float32),
    mesh=_mesh,
    scratch_types=[
        pltpu.VMEM((IDX_PER_W,), jnp.int32),
        pltpu.VMEM((NSLOT, PER_W, D), jnp.float32),
        pltpu.SemaphoreType.DMA,
        pltpu.SemaphoreType.DMA,
        pltpu.SemaphoreType.DMA,
        pltpu.SemaphoreType.DMA,
        pltpu.SemaphoreType.DMA,
        pltpu.SemaphoreType.DMA,
    ],
)
def _select_hands(x_hbm, out_hbm, idx_v, rowbuf,
                  sg0, sg1, sg2, so0, so1, so2):
    wid = lax.axis_index("s") * NC + lax.axis_index("c")
    wb = wid * PER_W
    sems_g = (sg0, sg1, sg2)
    sems_o = (so0, so1, so2)

    ramp = (wb + lax.iota(jnp.int32, 16)) * NROW
    def fill_idx(j):
        for k in range(PER_W // 16):
            idx_v[pl.ds((j * (PER_W // 16) + k) * 16, 16)] = (
                ramp + (k * 16 * NROW + _HANDS[j])
            )

    def start_gather(j, slot):
        return pltpu.async_copy(
            x_hbm.at[idx_v.at[pl.ds(j * PER_W, PER_W)]],
            rowbuf.at[slot],
            sems_g[slot],
        )

    def start_out(j, slot):
        return pltpu.async_copy(
            rowbuf.at[slot],
            out_hbm.at[j, pl.ds(wb, PER_W)],
            sems_o[slot],
        )

    gather_h = [None] * NSLOT
    out_h = [None] * NSLOT
    fill_idx(0)
    gather_h[0] = start_gather(0, 0)
    fill_idx(1)
    gather_h[1] = start_gather(1, 1)
    for j in range(NSEL):
        slot = j % NSLOT
        nslot = (j + 2) % NSLOT
        if j + 2 < NSEL:
            fill_idx(j + 2)
            if out_h[nslot] is not None:
                out_h[nslot].wait()
            gather_h[nslot] = start_gather(j + 2, nslot)
        gather_h[slot].wait()
        out_h[slot] = start_out(j, slot)
    for h in out_h:
        h.wait()


def kernel(input):
    x2d = input.reshape(B * NROW, D)
    planes = _select_hands(x2d)
    return jnp.swapaxes(planes, 0, 1)
